# baseline (speedup 1.0000x reference)
import functools

import jax
import jax.numpy as jnp
from jax import lax
from jax.experimental import pallas as pl
from jax.experimental.pallas import tpu as pltpu
from jax.experimental.pallas import tpu_sc as plsc

NC, NS, L = 2, 16, 16
N, E, D = 10000, 320000, 128
NP = 10240
RPT = NP // NS
W_IDX = 128

DEG_PER_TILE = E // (NC * NS)
DEG_PAD = -(-DEG_PER_TILE // W_IDX) * W_IDX
DEG_WIN = DEG_PAD // W_IDX

SCT_PER_TILE = E // NS
SCT_PAD = -(-SCT_PER_TILE // W_IDX) * W_IDX
SCT_WIN = SCT_PAD // W_IDX

BLK = 1000
GRID = N // BLK

@functools.cache
def _deg_kernel_fn():
  mesh = plsc.VectorSubcoreMesh(
      core_axis_name="c", subcore_axis_name="s", num_cores=NC, num_subcores=NS)

  @functools.partial(
      pl.kernel,
      out_type=jax.ShapeDtypeStruct((NC * NP,), jnp.float32),
      mesh=mesh,
      scratch_types=[
          pltpu.VMEM_SHARED((NP,), jnp.float32),
          pltpu.VMEM((W_IDX,), jnp.float32),
          pltpu.VMEM((DEG_WIN, W_IDX), jnp.int32),
      ],
  )
  def _deg_kernel(dst_hbm, ones_hbm, zeros_hbm, deg_hbm,
                  deg_sh, ones_v, idx_all):
    c = lax.axis_index("c")
    s = lax.axis_index("s")
    tile = c * NS + s
    pltpu.sync_copy(ones_hbm, ones_v)
    pltpu.sync_copy(dst_hbm.at[tile], idx_all)
    pltpu.sync_copy(zeros_hbm, deg_sh.at[pl.ds(s * RPT, RPT)])
    plsc.subcore_barrier()

    def win(w, carry):
        pltpu.sync_copy(ones_v, deg_sh.at[idx_all.at[w]], add=True)
        return carry

    lax.fori_loop(0, DEG_WIN, win, 0)
    plsc.subcore_barrier()
    pltpu.sync_copy(deg_sh.at[pl.ds(s * RPT, RPT)],
                    deg_hbm.at[pl.ds(c * NP + s * RPT, RPT)])

  return _deg_kernel


@functools.cache
def _msg_kernel_fn():
  mesh = plsc.VectorSubcoreMesh(
      core_axis_name="c", subcore_axis_name="s", num_cores=NC, num_subcores=NS)

  @functools.partial(
      pl.kernel,
      out_type=jax.ShapeDtypeStruct((NC * NP, D), jnp.float32),
      mesh=mesh,
      scratch_types=[
          pltpu.VMEM_SHARED((NP, D), jnp.float32),
          pltpu.VMEM((1, W_IDX), jnp.int32),
          pltpu.VMEM((1, W_IDX), jnp.int32),
          pltpu.VMEM((1, W_IDX), jnp.int32),
          pltpu.VMEM((1, W_IDX), jnp.int32),
          pltpu.VMEM((W_IDX, D), jnp.float32),
          pltpu.VMEM((W_IDX, D), jnp.float32),
          pltpu.SemaphoreType.DMA,
          pltpu.SemaphoreType.DMA,
          pltpu.SemaphoreType.DMA,
          pltpu.SemaphoreType.DMA,
      ],
  )
  def _msg_kernel(ys_hbm, src_hbm, dst_hbm, zeros_hbm, zt_hbm,
                  zt_sh, src_a, dst_a, src_b, dst_b,
                  rows_a, rows_b, sem_a, sem_b, sem_sa, sem_sb):
    c = lax.axis_index("c")
    s = lax.axis_index("s")
    tile = c * NS + s

    def zrow(i, carry):
        pltpu.sync_copy(zeros_hbm, zt_sh.at[pl.ds(s * RPT + i * 64, 64)])
        return carry

    lax.fori_loop(0, RPT // 64, zrow, 0)
    plsc.subcore_barrier()

    def wait_g(idx_ref, rows_ref, sem):
        pltpu.make_async_copy(ys_hbm.at[idx_ref.at[0]], rows_ref, sem).wait()

    def start_s(rows_ref, idx_ref, sem):
        pltpu.async_copy(rows_ref, zt_sh.at[idx_ref.at[0]], sem, add=True)

    def wait_s(rows_ref, idx_ref, sem):
        pltpu.make_async_copy(rows_ref, zt_sh.at[idx_ref.at[0]], sem).wait()

    pltpu.sync_copy(src_hbm.at[tile, pl.ds(0, 1)], src_a)
    pltpu.sync_copy(dst_hbm.at[tile, pl.ds(0, 1)], dst_a)
    pltpu.async_copy(ys_hbm.at[src_a.at[0]], rows_a, sem_a)
    pltpu.sync_copy(src_hbm.at[tile, pl.ds(1, 1)], src_b)
    pltpu.sync_copy(dst_hbm.at[tile, pl.ds(1, 1)], dst_b)
    pltpu.async_copy(ys_hbm.at[src_b.at[0]], rows_b, sem_b)

    def pipe(i, carry):
        wait_g(src_a, rows_a, sem_a)
        start_s(rows_a, dst_a, sem_sa)
        wait_g(src_b, rows_b, sem_b)
        start_s(rows_b, dst_b, sem_sb)
        wait_s(rows_a, dst_a, sem_sa)
        pltpu.sync_copy(src_hbm.at[tile, pl.ds(2 * i + 2, 1)], src_a)
        pltpu.sync_copy(dst_hbm.at[tile, pl.ds(2 * i + 2, 1)], dst_a)
        pltpu.async_copy(ys_hbm.at[src_a.at[0]], rows_a, sem_a)
        wait_s(rows_b, dst_b, sem_sb)
        pltpu.sync_copy(src_hbm.at[tile, pl.ds(2 * i + 3, 1)], src_b)
        pltpu.sync_copy(dst_hbm.at[tile, pl.ds(2 * i + 3, 1)], dst_b)
        pltpu.async_copy(ys_hbm.at[src_b.at[0]], rows_b, sem_b)
        return carry

    lax.fori_loop(0, (SCT_WIN - 3) // 2, pipe, 0)
    wait_g(src_a, rows_a, sem_a)
    start_s(rows_a, dst_a, sem_sa)
    wait_g(src_b, rows_b, sem_b)
    start_s(rows_b, dst_b, sem_sb)
    wait_s(rows_a, dst_a, sem_sa)
    pltpu.sync_copy(src_hbm.at[tile, pl.ds(SCT_WIN - 1, 1)], src_a)
    pltpu.sync_copy(dst_hbm.at[tile, pl.ds(SCT_WIN - 1, 1)], dst_a)
    pltpu.async_copy(ys_hbm.at[src_a.at[0]], rows_a, sem_a)
    wait_g(src_a, rows_a, sem_a)
    start_s(rows_a, dst_a, sem_sa)
    wait_s(rows_a, dst_a, sem_sa)
    wait_s(rows_b, dst_b, sem_sb)
    plsc.subcore_barrier()

    pltpu.sync_copy(zt_sh.at[pl.ds(s * RPT, RPT)],
                    zt_hbm.at[pl.ds(c * NP + s * RPT, RPT)])

  return _msg_kernel


def _y_body(seq1_ref, seq2_ref, w_ref, degs_ref, ys_ref, dinv_ref):
    deg = degs_ref[0] + degs_ref[1] + 1.0
    dinv = lax.rsqrt(deg)
    dinv_ref[...] = dinv
    w = w_ref[...]
    ys_ref[0] = jnp.dot(seq1_ref[...], w, preferred_element_type=jnp.float32) * dinv
    ys_ref[1] = jnp.dot(seq2_ref[...], w, preferred_element_type=jnp.float32) * dinv


def _y_call(seq1, seq2, w_gcn, degs):
    return pl.pallas_call(
        _y_body,
        grid=(GRID,),
        in_specs=[
            pl.BlockSpec((BLK, D), lambda g: (g, 0)),
            pl.BlockSpec((BLK, D), lambda g: (g, 0)),
            pl.BlockSpec((D, D), lambda g: (0, 0)),
            pl.BlockSpec((NC, BLK, 1), lambda g: (0, g, 0)),
        ],
        out_specs=[
            pl.BlockSpec((NC, BLK, D), lambda g: (0, g, 0)),
            pl.BlockSpec((BLK, 1), lambda g: (g, 0)),
        ],
        out_shape=[
            jax.ShapeDtypeStruct((NC, N, D), jnp.float32),
            jax.ShapeDtypeStruct((N, 1), jnp.float32),
        ],
    )(seq1, seq2, w_gcn, degs)


def _epi_body(zt_ref, ys_ref, dinv_ref, b_ref, a_ref, wd_ref, bd_ref,
              out_ref, h1_s, h2_s, sum_s):
    g = pl.program_id(0)
    a = a_ref[0, 0]
    dinv = dinv_ref[...]
    b = b_ref[...]
    z1 = dinv * (zt_ref[0] + ys_ref[0]) + b
    h1 = jnp.where(z1 >= 0, z1, a * z1)
    z2 = dinv * (zt_ref[1] + ys_ref[1]) + b
    h2 = jnp.where(z2 >= 0, z2, a * z2)
    h1_s[pl.ds(g * BLK, BLK)] = h1
    h2_s[pl.ds(g * BLK, BLK)] = h2

    @pl.when(g == 0)
    def _init():
        sum_s[...] = jnp.zeros_like(sum_s)

    sum_s[...] += jnp.sum(h1, axis=0, keepdims=True)

    @pl.when(g == pl.num_programs(0) - 1)
    def _finish():
        svec = jax.nn.sigmoid(sum_s[...] * (1.0 / N))
        cw = lax.dot_general(svec, wd_ref[...],
                             (((1,), (1,)), ((), ())),
                             preferred_element_type=jnp.float32)
        bd = bd_ref[0, 0]
        sc1 = lax.dot_general(cw, h1_s[...], (((1,), (1,)), ((), ())),
                              preferred_element_type=jnp.float32)
        sc2 = lax.dot_general(cw, h2_s[...], (((1,), (1,)), ((), ())),
                              preferred_element_type=jnp.float32)
        out_ref[0:1, :] = sc1 + bd
        out_ref[1:2, :] = sc2 + bd


def _epi_call(zt, ys, dinv, b_gcn, prelu_a, w_disc, b_disc):
    return pl.pallas_call(
        _epi_body,
        grid=(GRID,),
        in_specs=[
            pl.BlockSpec((NC, BLK, D), lambda g: (0, g, 0)),
            pl.BlockSpec((NC, BLK, D), lambda g: (0, g, 0)),
            pl.BlockSpec((BLK, 1), lambda g: (g, 0)),
            pl.BlockSpec((1, D), lambda g: (0, 0)),
            pl.BlockSpec(memory_space=pltpu.SMEM),
            pl.BlockSpec((D, D), lambda g: (0, 0)),
            pl.BlockSpec(memory_space=pltpu.SMEM),
        ],
        out_specs=pl.BlockSpec((NC, N), lambda g: (0, 0)),
        out_shape=jax.ShapeDtypeStruct((NC, N), jnp.float32),
        scratch_shapes=[
            pltpu.VMEM((N, D), jnp.float32),
            pltpu.VMEM((N, D), jnp.float32),
            pltpu.VMEM((1, D), jnp.float32),
        ],
    )(zt, ys, dinv, b_gcn, prelu_a, w_disc, b_disc)


def kernel(seq1, seq2, edge_index, W_gcn, b_gcn, prelu_a, W_disc, b_disc):
    src = edge_index[0]
    dst = edge_index[1]

    n_dummy = NP - N

    dstd = dst.reshape(NC * NS, DEG_PER_TILE)
    pad_d = N + (jnp.arange(DEG_PAD - DEG_PER_TILE, dtype=jnp.int32) % n_dummy)
    dstd = jnp.concatenate(
        [dstd, jnp.broadcast_to(pad_d, (NC * NS, DEG_PAD - DEG_PER_TILE))], axis=1)

    srcr = src.reshape(NS, SCT_PER_TILE)
    dstr = dst.reshape(NS, SCT_PER_TILE)
    pad_s = jnp.zeros((NS, SCT_PAD - SCT_PER_TILE), jnp.int32)
    pad_t = N + (jnp.arange(SCT_PAD - SCT_PER_TILE, dtype=jnp.int32) % n_dummy)
    srcr = jnp.concatenate([srcr, pad_s], axis=1)
    dstr = jnp.concatenate(
        [dstr, jnp.broadcast_to(pad_t, (NS, SCT_PAD - SCT_PER_TILE))], axis=1)
    srcs = jnp.concatenate([srcr, srcr + N], axis=0).reshape(NC * NS, SCT_WIN, W_IDX)
    dsts = jnp.concatenate([dstr, dstr], axis=0).reshape(NC * NS, SCT_WIN, W_IDX)
    dstd = dstd.reshape(NC * NS, DEG_WIN, W_IDX)

    ones_w = jnp.ones((W_IDX,), jnp.float32)
    zeros_r = jnp.zeros((RPT,), jnp.float32)
    zeros_rows = jnp.zeros((64, D), jnp.float32)

    deg_flat = _deg_kernel_fn()(dstd, ones_w, zeros_r)
    degs = deg_flat.reshape(NC, NP)[:, :N].reshape(NC, N, 1)

    ys, dinv = _y_call(seq1, seq2, W_gcn, degs)
    ys_flat = ys.reshape(NC * N, D)

    zt_flat = _msg_kernel_fn()(ys_flat, srcs, dsts, zeros_rows)
    zt = zt_flat.reshape(NC, NP, D)

    out2 = _epi_call(zt, ys, dinv, b_gcn.reshape(1, D), prelu_a.reshape(1, 1),
                     W_disc, b_disc.reshape(1, 1))
    return out2.reshape(1, 2 * N)


# speedup vs baseline: 1.0954x; 1.0954x over previous
import functools

import jax
import jax.numpy as jnp
from jax import lax
from jax.experimental import pallas as pl
from jax.experimental.pallas import tpu as pltpu
from jax.experimental.pallas import tpu_sc as plsc

NC, NS, L = 2, 16, 16
N, E, D = 10000, 320000, 128
NP = 10240
RPT = NP // NS
W_IDX = 128

DEG_PER_TILE = E // (NC * NS)
DEG_PAD = -(-DEG_PER_TILE // W_IDX) * W_IDX
DEG_WIN = DEG_PAD // W_IDX

SCT_PER_TILE = E // NS
BWIN = 16
SCT_PAD = -(-SCT_PER_TILE // (W_IDX * BWIN)) * W_IDX * BWIN
SCT_WIN = SCT_PAD // W_IDX

BLK = 1000
GRID = N // BLK

@functools.cache
def _deg_kernel_fn():
  mesh = plsc.VectorSubcoreMesh(
      core_axis_name="c", subcore_axis_name="s", num_cores=NC, num_subcores=NS)

  @functools.partial(
      pl.kernel,
      out_type=jax.ShapeDtypeStruct((NC * NP,), jnp.float32),
      mesh=mesh,
      scratch_types=[
          pltpu.VMEM_SHARED((NP,), jnp.float32),
          pltpu.VMEM((W_IDX,), jnp.float32),
          pltpu.VMEM((DEG_WIN, W_IDX), jnp.int32),
      ],
  )
  def _deg_kernel(dst_hbm, ones_hbm, zeros_hbm, deg_hbm,
                  deg_sh, ones_v, idx_all):
    c = lax.axis_index("c")
    s = lax.axis_index("s")
    tile = c * NS + s
    pltpu.sync_copy(ones_hbm, ones_v)
    pltpu.sync_copy(dst_hbm.at[tile], idx_all)
    pltpu.sync_copy(zeros_hbm, deg_sh.at[pl.ds(s * RPT, RPT)])
    plsc.subcore_barrier()

    def win(w, carry):
        pltpu.sync_copy(ones_v, deg_sh.at[idx_all.at[w]], add=True)
        return carry

    lax.fori_loop(0, DEG_WIN, win, 0)
    plsc.subcore_barrier()
    pltpu.sync_copy(deg_sh.at[pl.ds(s * RPT, RPT)],
                    deg_hbm.at[pl.ds(c * NP + s * RPT, RPT)])

  return _deg_kernel


@functools.cache
def _msg_kernel_fn():
  mesh = plsc.VectorSubcoreMesh(
      core_axis_name="c", subcore_axis_name="s", num_cores=NC, num_subcores=NS)

  @functools.partial(
      pl.kernel,
      out_type=jax.ShapeDtypeStruct((NC * NP, D), jnp.float32),
      mesh=mesh,
      scratch_types=[
          pltpu.VMEM_SHARED((NP, D), jnp.float32),
          pltpu.VMEM((BWIN, W_IDX), jnp.int32),
          pltpu.VMEM((BWIN, W_IDX), jnp.int32),
          pltpu.VMEM((W_IDX, D), jnp.float32),
          pltpu.VMEM((W_IDX, D), jnp.float32),
          pltpu.SemaphoreType.DMA,
          pltpu.SemaphoreType.DMA,
          pltpu.SemaphoreType.DMA,
          pltpu.SemaphoreType.DMA,
      ],
  )
  def _msg_kernel(ys_hbm, src_hbm, dst_hbm, zeros_hbm, zt_hbm,
                  zt_sh, src_blk, dst_blk,
                  rows_a, rows_b, sem_a, sem_b, sem_sa, sem_sb):
    c = lax.axis_index("c")
    s = lax.axis_index("s")
    tile = c * NS + s

    def zrow(i, carry):
        pltpu.sync_copy(zeros_hbm, zt_sh.at[pl.ds(s * RPT + i * 64, 64)])
        return carry

    lax.fori_loop(0, RPT // 64, zrow, 0)
    plsc.subcore_barrier()

    def fetch_blk(b):
        pltpu.sync_copy(src_hbm.at[tile, pl.ds(b * BWIN, BWIN)], src_blk)
        pltpu.sync_copy(dst_hbm.at[tile, pl.ds(b * BWIN, BWIN)], dst_blk)

    def start_g(j, rows_ref, sem):
        pltpu.async_copy(ys_hbm.at[src_blk.at[j]], rows_ref, sem)

    def wait_g(j, rows_ref, sem):
        pltpu.make_async_copy(ys_hbm.at[src_blk.at[j]], rows_ref, sem).wait()

    def start_s(rows_ref, j, sem):
        pltpu.async_copy(rows_ref, zt_sh.at[dst_blk.at[j]], sem, add=True)

    def wait_s(rows_ref, j, sem):
        pltpu.make_async_copy(rows_ref, zt_sh.at[dst_blk.at[j]], sem).wait()

    def block(b, carry):
        fetch_blk(b)
        start_g(0, rows_a, sem_a)
        start_g(1, rows_b, sem_b)

        def pipe(i, carry2):
            wait_g(2 * i, rows_a, sem_a)
            start_s(rows_a, 2 * i, sem_sa)
            wait_g(2 * i + 1, rows_b, sem_b)
            start_s(rows_b, 2 * i + 1, sem_sb)
            wait_s(rows_a, 2 * i, sem_sa)
            start_g(2 * i + 2, rows_a, sem_a)
            wait_s(rows_b, 2 * i + 1, sem_sb)
            start_g(2 * i + 3, rows_b, sem_b)
            return carry2

        lax.fori_loop(0, BWIN // 2 - 1, pipe, 0)
        wait_g(BWIN - 2, rows_a, sem_a)
        start_s(rows_a, BWIN - 2, sem_sa)
        wait_g(BWIN - 1, rows_b, sem_b)
        start_s(rows_b, BWIN - 1, sem_sb)
        wait_s(rows_a, BWIN - 2, sem_sa)
        wait_s(rows_b, BWIN - 1, sem_sb)
        return carry

    lax.fori_loop(0, SCT_WIN // BWIN, block, 0)
    plsc.subcore_barrier()

    pltpu.sync_copy(zt_sh.at[pl.ds(s * RPT, RPT)],
                    zt_hbm.at[pl.ds(c * NP + s * RPT, RPT)])

  return _msg_kernel


def _xw_body(seq1_ref, seq2_ref, w_ref, xw_ref):
    w = w_ref[...]
    xw_ref[0] = jnp.dot(seq1_ref[...], w, preferred_element_type=jnp.float32)
    xw_ref[1] = jnp.dot(seq2_ref[...], w, preferred_element_type=jnp.float32)


def _xw_call(seq1, seq2, w_gcn):
    return pl.pallas_call(
        _xw_body,
        grid=(GRID,),
        in_specs=[
            pl.BlockSpec((BLK, D), lambda g: (g, 0)),
            pl.BlockSpec((BLK, D), lambda g: (g, 0)),
            pl.BlockSpec((D, D), lambda g: (0, 0)),
        ],
        out_specs=pl.BlockSpec((NC, BLK, D), lambda g: (0, g, 0)),
        out_shape=jax.ShapeDtypeStruct((NC, N, D), jnp.float32),
    )(seq1, seq2, w_gcn)


def _y_body(xw_ref, degs_ref, ys_ref, dinv_ref):
    deg = degs_ref[0] + degs_ref[1] + 1.0
    dinv = lax.rsqrt(deg)
    dinv_ref[...] = dinv
    ys_ref[0] = xw_ref[0] * dinv
    ys_ref[1] = xw_ref[1] * dinv


def _y_call(xw, degs):
    return pl.pallas_call(
        _y_body,
        grid=(GRID,),
        in_specs=[
            pl.BlockSpec((NC, BLK, D), lambda g: (0, g, 0)),
            pl.BlockSpec((NC, BLK, 1), lambda g: (0, g, 0)),
        ],
        out_specs=[
            pl.BlockSpec((NC, BLK, D), lambda g: (0, g, 0)),
            pl.BlockSpec((BLK, 1), lambda g: (g, 0)),
        ],
        out_shape=[
            jax.ShapeDtypeStruct((NC, N, D), jnp.float32),
            jax.ShapeDtypeStruct((N, 1), jnp.float32),
        ],
    )(xw, degs)


def _epi_body(zt_ref, ys_ref, dinv_ref, b_ref, a_ref, wd_ref, bd_ref,
              out_ref, h1_s, h2_s, sum_s):
    g = pl.program_id(0)
    a = a_ref[0, 0]
    dinv = dinv_ref[...]
    b = b_ref[...]
    z1 = dinv * (zt_ref[0] + ys_ref[0]) + b
    h1 = jnp.where(z1 >= 0, z1, a * z1)
    z2 = dinv * (zt_ref[1] + ys_ref[1]) + b
    h2 = jnp.where(z2 >= 0, z2, a * z2)
    h1_s[pl.ds(g * BLK, BLK)] = h1
    h2_s[pl.ds(g * BLK, BLK)] = h2

    @pl.when(g == 0)
    def _init():
        sum_s[...] = jnp.zeros_like(sum_s)

    sum_s[...] += jnp.sum(h1, axis=0, keepdims=True)

    @pl.when(g == pl.num_programs(0) - 1)
    def _finish():
        svec = jax.nn.sigmoid(sum_s[...] * (1.0 / N))
        cw = lax.dot_general(svec, wd_ref[...],
                             (((1,), (1,)), ((), ())),
                             preferred_element_type=jnp.float32)
        bd = bd_ref[0, 0]
        sc1 = lax.dot_general(cw, h1_s[...], (((1,), (1,)), ((), ())),
                              preferred_element_type=jnp.float32)
        sc2 = lax.dot_general(cw, h2_s[...], (((1,), (1,)), ((), ())),
                              preferred_element_type=jnp.float32)
        out_ref[0:1, :] = sc1 + bd
        out_ref[1:2, :] = sc2 + bd


def _epi_call(zt, ys, dinv, b_gcn, prelu_a, w_disc, b_disc):
    return pl.pallas_call(
        _epi_body,
        grid=(GRID,),
        in_specs=[
            pl.BlockSpec((NC, BLK, D), lambda g: (0, g, 0)),
            pl.BlockSpec((NC, BLK, D), lambda g: (0, g, 0)),
            pl.BlockSpec((BLK, 1), lambda g: (g, 0)),
            pl.BlockSpec((1, D), lambda g: (0, 0)),
            pl.BlockSpec(memory_space=pltpu.SMEM),
            pl.BlockSpec((D, D), lambda g: (0, 0)),
            pl.BlockSpec(memory_space=pltpu.SMEM),
        ],
        out_specs=pl.BlockSpec((NC, N), lambda g: (0, 0)),
        out_shape=jax.ShapeDtypeStruct((NC, N), jnp.float32),
        scratch_shapes=[
            pltpu.VMEM((N, D), jnp.float32),
            pltpu.VMEM((N, D), jnp.float32),
            pltpu.VMEM((1, D), jnp.float32),
        ],
    )(zt, ys, dinv, b_gcn, prelu_a, w_disc, b_disc)


def kernel(seq1, seq2, edge_index, W_gcn, b_gcn, prelu_a, W_disc, b_disc):
    src = edge_index[0]
    dst = edge_index[1]

    n_dummy = NP - N

    dstd = dst.reshape(NC * NS, DEG_PER_TILE)
    pad_d = N + (jnp.arange(DEG_PAD - DEG_PER_TILE, dtype=jnp.int32) % n_dummy)
    dstd = jnp.concatenate(
        [dstd, jnp.broadcast_to(pad_d, (NC * NS, DEG_PAD - DEG_PER_TILE))], axis=1)

    srcr = src.reshape(NS, SCT_PER_TILE)
    dstr = dst.reshape(NS, SCT_PER_TILE)
    npad = SCT_PAD - SCT_PER_TILE
    pad_s = jnp.broadcast_to((jnp.arange(npad, dtype=jnp.int32) * 131) % N,
                             (NS, npad))
    pad_t = N + (jnp.arange(npad, dtype=jnp.int32) % n_dummy)
    srcr = jnp.concatenate([srcr, pad_s], axis=1)
    dstr = jnp.concatenate(
        [dstr, jnp.broadcast_to(pad_t, (NS, SCT_PAD - SCT_PER_TILE))], axis=1)
    srcs = jnp.concatenate([srcr, srcr + N], axis=0).reshape(NC * NS, SCT_WIN, W_IDX)
    dsts = jnp.concatenate([dstr, dstr], axis=0).reshape(NC * NS, SCT_WIN, W_IDX)
    dstd = dstd.reshape(NC * NS, DEG_WIN, W_IDX)

    ones_w = jnp.ones((W_IDX,), jnp.float32)
    zeros_r = jnp.zeros((RPT,), jnp.float32)
    zeros_rows = jnp.zeros((64, D), jnp.float32)

    xw = _xw_call(seq1, seq2, W_gcn)
    deg_flat = _deg_kernel_fn()(dstd, ones_w, zeros_r)
    degs = deg_flat.reshape(NC, NP)[:, :N].reshape(NC, N, 1)

    ys, dinv = _y_call(xw, degs)
    ys_flat = ys.reshape(NC * N, D)

    zt_flat = _msg_kernel_fn()(ys_flat, srcs, dsts, zeros_rows)
    zt = zt_flat.reshape(NC, NP, D)

    out2 = _epi_call(zt, ys, dinv, b_gcn.reshape(1, D), prelu_a.reshape(1, 1),
                     W_disc, b_disc.reshape(1, 1))
    return out2.reshape(1, 2 * N)


# speedup vs baseline: 1.1187x; 1.0213x over previous
import functools

import jax
import jax.numpy as jnp
from jax import lax
from jax.experimental import pallas as pl
from jax.experimental.pallas import tpu as pltpu
from jax.experimental.pallas import tpu_sc as plsc

NC, NS, L = 2, 16, 16
N, E, D = 10000, 320000, 128
NP = 10240
RPT = NP // NS
W_IDX = 128

DEG_PER_TILE = E // (NC * NS)
DEG_PAD = -(-DEG_PER_TILE // W_IDX) * W_IDX
DEG_WIN = DEG_PAD // W_IDX

SCT_PER_TILE = E // NS
BWIN = 40
SCT_PAD = -(-SCT_PER_TILE // (W_IDX * BWIN)) * W_IDX * BWIN
SCT_WIN = SCT_PAD // W_IDX

BLK = 1000
GRID = N // BLK

@functools.cache
def _deg_kernel_fn():
  mesh = plsc.VectorSubcoreMesh(
      core_axis_name="c", subcore_axis_name="s", num_cores=NC, num_subcores=NS)

  @functools.partial(
      pl.kernel,
      out_type=jax.ShapeDtypeStruct((NC * NP,), jnp.float32),
      mesh=mesh,
      scratch_types=[
          pltpu.VMEM_SHARED((NP,), jnp.float32),
          pltpu.VMEM((W_IDX,), jnp.float32),
          pltpu.VMEM((DEG_WIN, W_IDX), jnp.int32),
      ],
  )
  def _deg_kernel(dst_hbm, ones_hbm, zeros_hbm, deg_hbm,
                  deg_sh, ones_v, idx_all):
    c = lax.axis_index("c")
    s = lax.axis_index("s")
    tile = c * NS + s
    pltpu.sync_copy(ones_hbm, ones_v)
    pltpu.sync_copy(dst_hbm.at[tile], idx_all)
    pltpu.sync_copy(zeros_hbm, deg_sh.at[pl.ds(s * RPT, RPT)])
    plsc.subcore_barrier()

    def win(w, carry):
        pltpu.sync_copy(ones_v, deg_sh.at[idx_all.at[w]], add=True)
        return carry

    lax.fori_loop(0, DEG_WIN, win, 0)
    plsc.subcore_barrier()
    pltpu.sync_copy(deg_sh.at[pl.ds(s * RPT, RPT)],
                    deg_hbm.at[pl.ds(c * NP + s * RPT, RPT)])

  return _deg_kernel


@functools.cache
def _msg_kernel_fn():
  mesh = plsc.VectorSubcoreMesh(
      core_axis_name="c", subcore_axis_name="s", num_cores=NC, num_subcores=NS)

  @functools.partial(
      pl.kernel,
      out_type=jax.ShapeDtypeStruct((NC * NP, D), jnp.float32),
      mesh=mesh,
      scratch_types=[
          pltpu.VMEM_SHARED((NP, D), jnp.float32),
          pltpu.VMEM((BWIN, W_IDX), jnp.int32),
          pltpu.VMEM((BWIN, W_IDX), jnp.int32),
          pltpu.VMEM((W_IDX, D), jnp.float32),
          pltpu.VMEM((W_IDX, D), jnp.float32),
          pltpu.SemaphoreType.DMA,
          pltpu.SemaphoreType.DMA,
          pltpu.SemaphoreType.DMA,
          pltpu.SemaphoreType.DMA,
      ],
  )
  def _msg_kernel(ys_hbm, src_hbm, dst_hbm, zeros_hbm, zt_hbm,
                  zt_sh, src_blk, dst_blk,
                  rows_a, rows_b, sem_a, sem_b, sem_sa, sem_sb):
    c = lax.axis_index("c")
    s = lax.axis_index("s")
    tile = c * NS + s

    def zrow(i, carry):
        pltpu.sync_copy(zeros_hbm, zt_sh.at[pl.ds(s * RPT + i * 64, 64)])
        return carry

    lax.fori_loop(0, RPT // 64, zrow, 0)
    plsc.subcore_barrier()

    def fetch_blk(b):
        pltpu.sync_copy(src_hbm.at[tile, pl.ds(b * BWIN, BWIN)], src_blk)
        pltpu.sync_copy(dst_hbm.at[tile, pl.ds(b * BWIN, BWIN)], dst_blk)

    def start_g(j, rows_ref, sem):
        pltpu.async_copy(ys_hbm.at[src_blk.at[j]], rows_ref, sem)

    def wait_g(j, rows_ref, sem):
        pltpu.make_async_copy(ys_hbm.at[src_blk.at[j]], rows_ref, sem).wait()

    def start_s(rows_ref, j, sem):
        pltpu.async_copy(rows_ref, zt_sh.at[dst_blk.at[j]], sem, add=True)

    def wait_s(rows_ref, j, sem):
        pltpu.make_async_copy(rows_ref, zt_sh.at[dst_blk.at[j]], sem).wait()

    def block(b, carry):
        fetch_blk(b)
        start_g(0, rows_a, sem_a)
        start_g(1, rows_b, sem_b)

        def pipe(i, carry2):
            wait_g(2 * i, rows_a, sem_a)
            start_s(rows_a, 2 * i, sem_sa)
            wait_g(2 * i + 1, rows_b, sem_b)
            start_s(rows_b, 2 * i + 1, sem_sb)
            wait_s(rows_a, 2 * i, sem_sa)
            start_g(2 * i + 2, rows_a, sem_a)
            wait_s(rows_b, 2 * i + 1, sem_sb)
            start_g(2 * i + 3, rows_b, sem_b)
            return carry2

        lax.fori_loop(0, BWIN // 2 - 1, pipe, 0)
        wait_g(BWIN - 2, rows_a, sem_a)
        start_s(rows_a, BWIN - 2, sem_sa)
        wait_g(BWIN - 1, rows_b, sem_b)
        start_s(rows_b, BWIN - 1, sem_sb)
        wait_s(rows_a, BWIN - 2, sem_sa)
        wait_s(rows_b, BWIN - 1, sem_sb)
        return carry

    lax.fori_loop(0, SCT_WIN // BWIN, block, 0)
    plsc.subcore_barrier()

    pltpu.sync_copy(zt_sh.at[pl.ds(s * RPT, RPT)],
                    zt_hbm.at[pl.ds(c * NP + s * RPT, RPT)])

  return _msg_kernel


def _xw_body(seq1_ref, seq2_ref, w_ref, xw_ref):
    w = w_ref[...]
    xw_ref[0] = jnp.dot(seq1_ref[...], w, preferred_element_type=jnp.float32)
    xw_ref[1] = jnp.dot(seq2_ref[...], w, preferred_element_type=jnp.float32)


def _xw_call(seq1, seq2, w_gcn):
    return pl.pallas_call(
        _xw_body,
        grid=(GRID,),
        in_specs=[
            pl.BlockSpec((BLK, D), lambda g: (g, 0)),
            pl.BlockSpec((BLK, D), lambda g: (g, 0)),
            pl.BlockSpec((D, D), lambda g: (0, 0)),
        ],
        out_specs=pl.BlockSpec((NC, BLK, D), lambda g: (0, g, 0)),
        out_shape=jax.ShapeDtypeStruct((NC, N, D), jnp.float32),
    )(seq1, seq2, w_gcn)


def _y_body(xw_ref, degs_ref, ys_ref, dinv_ref):
    deg = degs_ref[0] + degs_ref[1] + 1.0
    dinv = lax.rsqrt(deg)
    dinv_ref[...] = dinv
    ys_ref[0] = xw_ref[0] * dinv
    ys_ref[1] = xw_ref[1] * dinv


def _y_call(xw, degs):
    return pl.pallas_call(
        _y_body,
        grid=(GRID,),
        in_specs=[
            pl.BlockSpec((NC, BLK, D), lambda g: (0, g, 0)),
            pl.BlockSpec((NC, BLK, 1), lambda g: (0, g, 0)),
        ],
        out_specs=[
            pl.BlockSpec((NC, BLK, D), lambda g: (0, g, 0)),
            pl.BlockSpec((BLK, 1), lambda g: (g, 0)),
        ],
        out_shape=[
            jax.ShapeDtypeStruct((NC, N, D), jnp.float32),
            jax.ShapeDtypeStruct((N, 1), jnp.float32),
        ],
    )(xw, degs)


def _epi_body(zt_ref, ys_ref, dinv_ref, b_ref, a_ref, wd_ref, bd_ref,
              out_ref, h1_s, h2_s, sum_s):
    g = pl.program_id(0)
    a = a_ref[0, 0]
    dinv = dinv_ref[...]
    b = b_ref[...]
    z1 = dinv * (zt_ref[0] + ys_ref[0]) + b
    h1 = jnp.where(z1 >= 0, z1, a * z1)
    z2 = dinv * (zt_ref[1] + ys_ref[1]) + b
    h2 = jnp.where(z2 >= 0, z2, a * z2)
    h1_s[pl.ds(g * BLK, BLK)] = h1
    h2_s[pl.ds(g * BLK, BLK)] = h2

    @pl.when(g == 0)
    def _init():
        sum_s[...] = jnp.zeros_like(sum_s)

    sum_s[...] += jnp.sum(h1, axis=0, keepdims=True)

    @pl.when(g == pl.num_programs(0) - 1)
    def _finish():
        svec = jax.nn.sigmoid(sum_s[...] * (1.0 / N))
        cw = lax.dot_general(svec, wd_ref[...],
                             (((1,), (1,)), ((), ())),
                             preferred_element_type=jnp.float32)
        bd = bd_ref[0, 0]
        sc1 = lax.dot_general(cw, h1_s[...], (((1,), (1,)), ((), ())),
                              preferred_element_type=jnp.float32)
        sc2 = lax.dot_general(cw, h2_s[...], (((1,), (1,)), ((), ())),
                              preferred_element_type=jnp.float32)
        out_ref[0:1, :] = sc1 + bd
        out_ref[1:2, :] = sc2 + bd


def _epi_call(zt, ys, dinv, b_gcn, prelu_a, w_disc, b_disc):
    return pl.pallas_call(
        _epi_body,
        grid=(GRID,),
        in_specs=[
            pl.BlockSpec((NC, BLK, D), lambda g: (0, g, 0)),
            pl.BlockSpec((NC, BLK, D), lambda g: (0, g, 0)),
            pl.BlockSpec((BLK, 1), lambda g: (g, 0)),
            pl.BlockSpec((1, D), lambda g: (0, 0)),
            pl.BlockSpec(memory_space=pltpu.SMEM),
            pl.BlockSpec((D, D), lambda g: (0, 0)),
            pl.BlockSpec(memory_space=pltpu.SMEM),
        ],
        out_specs=pl.BlockSpec((NC, N), lambda g: (0, 0)),
        out_shape=jax.ShapeDtypeStruct((NC, N), jnp.float32),
        scratch_shapes=[
            pltpu.VMEM((N, D), jnp.float32),
            pltpu.VMEM((N, D), jnp.float32),
            pltpu.VMEM((1, D), jnp.float32),
        ],
    )(zt, ys, dinv, b_gcn, prelu_a, w_disc, b_disc)


def kernel(seq1, seq2, edge_index, W_gcn, b_gcn, prelu_a, W_disc, b_disc):
    src = edge_index[0]
    dst = edge_index[1]

    n_dummy = NP - N

    dstd = dst.reshape(NC * NS, DEG_PER_TILE)
    pad_d = N + (jnp.arange(DEG_PAD - DEG_PER_TILE, dtype=jnp.int32) % n_dummy)
    dstd = jnp.concatenate(
        [dstd, jnp.broadcast_to(pad_d, (NC * NS, DEG_PAD - DEG_PER_TILE))], axis=1)

    srcr = src.reshape(NS, SCT_PER_TILE)
    dstr = dst.reshape(NS, SCT_PER_TILE)
    npad = SCT_PAD - SCT_PER_TILE
    pad_s = jnp.broadcast_to((jnp.arange(npad, dtype=jnp.int32) * 131) % N,
                             (NS, npad))
    pad_t = N + (jnp.arange(npad, dtype=jnp.int32) % n_dummy)
    srcr = jnp.concatenate([srcr, pad_s], axis=1)
    dstr = jnp.concatenate(
        [dstr, jnp.broadcast_to(pad_t, (NS, SCT_PAD - SCT_PER_TILE))], axis=1)
    srcs = jnp.concatenate([srcr, srcr + N], axis=0).reshape(NC * NS, SCT_WIN, W_IDX)
    dsts = jnp.concatenate([dstr, dstr], axis=0).reshape(NC * NS, SCT_WIN, W_IDX)
    dstd = dstd.reshape(NC * NS, DEG_WIN, W_IDX)

    ones_w = jnp.ones((W_IDX,), jnp.float32)
    zeros_r = jnp.zeros((RPT,), jnp.float32)
    zeros_rows = jnp.zeros((64, D), jnp.float32)

    xw = _xw_call(seq1, seq2, W_gcn)
    deg_flat = _deg_kernel_fn()(dstd, ones_w, zeros_r)
    degs = deg_flat.reshape(NC, NP)[:, :N].reshape(NC, N, 1)

    ys, dinv = _y_call(xw, degs)
    ys_flat = ys.reshape(NC * N, D)

    zt_flat = _msg_kernel_fn()(ys_flat, srcs, dsts, zeros_rows)
    zt = zt_flat.reshape(NC, NP, D)

    out2 = _epi_call(zt, ys, dinv, b_gcn.reshape(1, D), prelu_a.reshape(1, 1),
                     W_disc, b_disc.reshape(1, 1))
    return out2.reshape(1, 2 * N)


# speedup vs baseline: 1.1239x; 1.0046x over previous
import functools

import jax
import jax.numpy as jnp
from jax import lax
from jax.experimental import pallas as pl
from jax.experimental.pallas import tpu as pltpu
from jax.experimental.pallas import tpu_sc as plsc

NC, NS, L = 2, 16, 16
N, E, D = 10000, 320000, 128
NP = 10240
RPT = NP // NS
W_IDX = 128

DEG_PER_TILE = E // (NC * NS)
DEG_PAD = -(-DEG_PER_TILE // (W_IDX * 8)) * W_IDX * 8
DEG_WIN = DEG_PAD // W_IDX

SCT_PER_TILE = E // NS
BWIN = 40
SCT_PAD = -(-SCT_PER_TILE // (W_IDX * BWIN)) * W_IDX * BWIN
SCT_WIN = SCT_PAD // W_IDX

BLK = 1000
GRID = N // BLK

@functools.cache
def _deg_kernel_fn():
  mesh = plsc.VectorSubcoreMesh(
      core_axis_name="c", subcore_axis_name="s", num_cores=NC, num_subcores=NS)

  @functools.partial(
      pl.kernel,
      out_type=jax.ShapeDtypeStruct((NC * NP,), jnp.float32),
      mesh=mesh,
      scratch_types=[
          pltpu.VMEM_SHARED((NP,), jnp.float32),
          pltpu.VMEM((W_IDX,), jnp.float32),
          pltpu.VMEM((DEG_WIN, W_IDX), jnp.int32),
          pltpu.SemaphoreType.DMA,
      ],
  )
  def _deg_kernel(dst_hbm, ones_hbm, zeros_hbm, deg_hbm,
                  deg_sh, ones_v, idx_all, dsem):
    c = lax.axis_index("c")
    s = lax.axis_index("s")
    tile = c * NS + s
    pltpu.sync_copy(ones_hbm, ones_v)
    pltpu.sync_copy(dst_hbm.at[tile], idx_all)
    pltpu.sync_copy(zeros_hbm, deg_sh.at[pl.ds(s * RPT, RPT)])
    plsc.subcore_barrier()

    def batch(bi, carry):
        for k in range(8):
            pltpu.async_copy(ones_v, deg_sh.at[idx_all.at[8 * bi + k]], dsem,
                             add=True)
        for k in range(8):
            pltpu.make_async_copy(ones_v, deg_sh.at[idx_all.at[8 * bi + k]],
                                  dsem).wait()
        return carry

    lax.fori_loop(0, DEG_WIN // 8, batch, 0)
    plsc.subcore_barrier()
    pltpu.sync_copy(deg_sh.at[pl.ds(s * RPT, RPT)],
                    deg_hbm.at[pl.ds(c * NP + s * RPT, RPT)])

  return _deg_kernel


@functools.cache
def _msg_kernel_fn():
  mesh = plsc.VectorSubcoreMesh(
      core_axis_name="c", subcore_axis_name="s", num_cores=NC, num_subcores=NS)

  @functools.partial(
      pl.kernel,
      out_type=jax.ShapeDtypeStruct((NC * NP, D), jnp.float32),
      mesh=mesh,
      scratch_types=[
          pltpu.VMEM_SHARED((NP, D), jnp.float32),
          pltpu.VMEM((BWIN, W_IDX), jnp.int32),
          pltpu.VMEM((BWIN, W_IDX), jnp.int32),
          pltpu.VMEM((W_IDX, D), jnp.float32),
          pltpu.VMEM((W_IDX, D), jnp.float32),
          pltpu.SemaphoreType.DMA,
          pltpu.SemaphoreType.DMA,
          pltpu.SemaphoreType.DMA,
          pltpu.SemaphoreType.DMA,
      ],
  )
  def _msg_kernel(ys_hbm, src_hbm, dst_hbm, zeros_hbm, zt_hbm,
                  zt_sh, src_blk, dst_blk,
                  rows_a, rows_b, sem_a, sem_b, sem_sa, sem_sb):
    c = lax.axis_index("c")
    s = lax.axis_index("s")
    tile = c * NS + s

    def zrow(i, carry):
        pltpu.sync_copy(zeros_hbm, zt_sh.at[pl.ds(s * RPT + i * 64, 64)])
        return carry

    lax.fori_loop(0, RPT // 64, zrow, 0)
    plsc.subcore_barrier()

    def fetch_blk(b):
        pltpu.sync_copy(src_hbm.at[tile, pl.ds(b * BWIN, BWIN)], src_blk)
        pltpu.sync_copy(dst_hbm.at[tile, pl.ds(b * BWIN, BWIN)], dst_blk)

    def start_g(j, rows_ref, sem):
        pltpu.async_copy(ys_hbm.at[src_blk.at[j]], rows_ref, sem)

    def wait_g(j, rows_ref, sem):
        pltpu.make_async_copy(ys_hbm.at[src_blk.at[j]], rows_ref, sem).wait()

    def start_s(rows_ref, j, sem):
        pltpu.async_copy(rows_ref, zt_sh.at[dst_blk.at[j]], sem, add=True)

    def wait_s(rows_ref, j, sem):
        pltpu.make_async_copy(rows_ref, zt_sh.at[dst_blk.at[j]], sem).wait()

    def block(b, carry):
        fetch_blk(b)
        start_g(0, rows_a, sem_a)
        start_g(1, rows_b, sem_b)

        def pipe(i, carry2):
            wait_g(2 * i, rows_a, sem_a)
            start_s(rows_a, 2 * i, sem_sa)
            wait_g(2 * i + 1, rows_b, sem_b)
            start_s(rows_b, 2 * i + 1, sem_sb)
            wait_s(rows_a, 2 * i, sem_sa)
            start_g(2 * i + 2, rows_a, sem_a)
            wait_s(rows_b, 2 * i + 1, sem_sb)
            start_g(2 * i + 3, rows_b, sem_b)
            return carry2

        lax.fori_loop(0, BWIN // 2 - 1, pipe, 0)
        wait_g(BWIN - 2, rows_a, sem_a)
        start_s(rows_a, BWIN - 2, sem_sa)
        wait_g(BWIN - 1, rows_b, sem_b)
        start_s(rows_b, BWIN - 1, sem_sb)
        wait_s(rows_a, BWIN - 2, sem_sa)
        wait_s(rows_b, BWIN - 1, sem_sb)
        return carry

    lax.fori_loop(0, SCT_WIN // BWIN, block, 0)
    plsc.subcore_barrier()

    pltpu.sync_copy(zt_sh.at[pl.ds(s * RPT, RPT)],
                    zt_hbm.at[pl.ds(c * NP + s * RPT, RPT)])

  return _msg_kernel


def _xw_body(seq1_ref, seq2_ref, w_ref, xw_ref):
    w = w_ref[...]
    xw_ref[0] = jnp.dot(seq1_ref[...], w, preferred_element_type=jnp.float32)
    xw_ref[1] = jnp.dot(seq2_ref[...], w, preferred_element_type=jnp.float32)


def _xw_call(seq1, seq2, w_gcn):
    return pl.pallas_call(
        _xw_body,
        grid=(GRID,),
        in_specs=[
            pl.BlockSpec((BLK, D), lambda g: (g, 0)),
            pl.BlockSpec((BLK, D), lambda g: (g, 0)),
            pl.BlockSpec((D, D), lambda g: (0, 0)),
        ],
        out_specs=pl.BlockSpec((NC, BLK, D), lambda g: (0, g, 0)),
        out_shape=jax.ShapeDtypeStruct((NC, N, D), jnp.float32),
    )(seq1, seq2, w_gcn)


def _y_body(xw_ref, degs_ref, ys_ref, dinv_ref):
    deg = degs_ref[0] + degs_ref[1] + 1.0
    dinv = lax.rsqrt(deg)
    dinv_ref[...] = dinv
    ys_ref[0] = xw_ref[0] * dinv
    ys_ref[1] = xw_ref[1] * dinv


def _y_call(xw, degs):
    return pl.pallas_call(
        _y_body,
        grid=(GRID,),
        in_specs=[
            pl.BlockSpec((NC, BLK, D), lambda g: (0, g, 0)),
            pl.BlockSpec((NC, BLK, 1), lambda g: (0, g, 0)),
        ],
        out_specs=[
            pl.BlockSpec((NC, BLK, D), lambda g: (0, g, 0)),
            pl.BlockSpec((BLK, 1), lambda g: (g, 0)),
        ],
        out_shape=[
            jax.ShapeDtypeStruct((NC, N, D), jnp.float32),
            jax.ShapeDtypeStruct((N, 1), jnp.float32),
        ],
    )(xw, degs)


def _epi_body(zt_ref, ys_ref, dinv_ref, b_ref, a_ref, wd_ref, bd_ref,
              out_ref, h1_s, h2_s, sum_s):
    g = pl.program_id(0)
    a = a_ref[0, 0]
    dinv = dinv_ref[...]
    b = b_ref[...]
    z1 = dinv * (zt_ref[0] + ys_ref[0]) + b
    h1 = jnp.where(z1 >= 0, z1, a * z1)
    z2 = dinv * (zt_ref[1] + ys_ref[1]) + b
    h2 = jnp.where(z2 >= 0, z2, a * z2)
    h1_s[pl.ds(g * BLK, BLK)] = h1
    h2_s[pl.ds(g * BLK, BLK)] = h2

    @pl.when(g == 0)
    def _init():
        sum_s[...] = jnp.zeros_like(sum_s)

    sum_s[...] += jnp.sum(h1, axis=0, keepdims=True)

    @pl.when(g == pl.num_programs(0) - 1)
    def _finish():
        svec = jax.nn.sigmoid(sum_s[...] * (1.0 / N))
        cw = lax.dot_general(svec, wd_ref[...],
                             (((1,), (1,)), ((), ())),
                             preferred_element_type=jnp.float32)
        bd = bd_ref[0, 0]
        sc1 = lax.dot_general(cw, h1_s[...], (((1,), (1,)), ((), ())),
                              preferred_element_type=jnp.float32)
        sc2 = lax.dot_general(cw, h2_s[...], (((1,), (1,)), ((), ())),
                              preferred_element_type=jnp.float32)
        out_ref[0:1, :] = sc1 + bd
        out_ref[1:2, :] = sc2 + bd


def _epi_call(zt, ys, dinv, b_gcn, prelu_a, w_disc, b_disc):
    return pl.pallas_call(
        _epi_body,
        grid=(GRID,),
        in_specs=[
            pl.BlockSpec((NC, BLK, D), lambda g: (0, g, 0)),
            pl.BlockSpec((NC, BLK, D), lambda g: (0, g, 0)),
            pl.BlockSpec((BLK, 1), lambda g: (g, 0)),
            pl.BlockSpec((1, D), lambda g: (0, 0)),
            pl.BlockSpec(memory_space=pltpu.SMEM),
            pl.BlockSpec((D, D), lambda g: (0, 0)),
            pl.BlockSpec(memory_space=pltpu.SMEM),
        ],
        out_specs=pl.BlockSpec((NC, N), lambda g: (0, 0)),
        out_shape=jax.ShapeDtypeStruct((NC, N), jnp.float32),
        scratch_shapes=[
            pltpu.VMEM((N, D), jnp.float32),
            pltpu.VMEM((N, D), jnp.float32),
            pltpu.VMEM((1, D), jnp.float32),
        ],
    )(zt, ys, dinv, b_gcn, prelu_a, w_disc, b_disc)


def kernel(seq1, seq2, edge_index, W_gcn, b_gcn, prelu_a, W_disc, b_disc):
    src = edge_index[0]
    dst = edge_index[1]

    n_dummy = NP - N

    dstd = dst.reshape(NC * NS, DEG_PER_TILE)
    pad_d = N + (jnp.arange(DEG_PAD - DEG_PER_TILE, dtype=jnp.int32) % n_dummy)
    dstd = jnp.concatenate(
        [dstd, jnp.broadcast_to(pad_d, (NC * NS, DEG_PAD - DEG_PER_TILE))], axis=1)

    srcr = src.reshape(NS, SCT_PER_TILE)
    dstr = dst.reshape(NS, SCT_PER_TILE)
    npad = SCT_PAD - SCT_PER_TILE
    pad_s = jnp.broadcast_to((jnp.arange(npad, dtype=jnp.int32) * 131) % N,
                             (NS, npad))
    pad_t = N + (jnp.arange(npad, dtype=jnp.int32) % n_dummy)
    srcr = jnp.concatenate([srcr, pad_s], axis=1)
    dstr = jnp.concatenate(
        [dstr, jnp.broadcast_to(pad_t, (NS, SCT_PAD - SCT_PER_TILE))], axis=1)
    srcs = jnp.concatenate([srcr, srcr + N], axis=0).reshape(NC * NS, SCT_WIN, W_IDX)
    dsts = jnp.concatenate([dstr, dstr], axis=0).reshape(NC * NS, SCT_WIN, W_IDX)
    dstd = dstd.reshape(NC * NS, DEG_WIN, W_IDX)

    ones_w = jnp.ones((W_IDX,), jnp.float32)
    zeros_r = jnp.zeros((RPT,), jnp.float32)
    zeros_rows = jnp.zeros((64, D), jnp.float32)

    xw = _xw_call(seq1, seq2, W_gcn)
    deg_flat = _deg_kernel_fn()(dstd, ones_w, zeros_r)
    degs = deg_flat.reshape(NC, NP)[:, :N].reshape(NC, N, 1)

    ys, dinv = _y_call(xw, degs)
    ys_flat = ys.reshape(NC * N, D)

    zt_flat = _msg_kernel_fn()(ys_flat, srcs, dsts, zeros_rows)
    zt = zt_flat.reshape(NC, NP, D)

    out2 = _epi_call(zt, ys, dinv, b_gcn.reshape(1, D), prelu_a.reshape(1, 1),
                     W_disc, b_disc.reshape(1, 1))
    return out2.reshape(1, 2 * N)
